# baseline (speedup 1.0000x reference)
import numpy as np
import jax
import jax.numpy as jnp
from jax.experimental import pallas as pl
from jax.experimental.pallas import tpu as pltpu

_N = 512
_D = 4096
_CHUNK = 512
_NCHUNKS = _D // _CHUNK
_SLOTS = 3


def _triplet_columns(n):
    labels = list(range(int(n / 2))) + list(range(int(n / 2)))
    rng = np.random.RandomState(0)
    triplets = []
    for i in range(len(labels)):
        triplets_i = []
        for j in range(len(labels)):
            if labels[i] == labels[j] and i != j:
                for k in range(len(labels)):
                    if labels[i] != labels[k]:
                        triplets_i.append([i, j, k])
        rng.shuffle(triplets_i)
        triplets += triplets_i[:3]
    trip = np.asarray(triplets, dtype=np.int32)
    jj = np.zeros((_SLOTS, n), dtype=np.int32)
    kk = np.zeros((_SLOTS, n), dtype=np.int32)
    fill = np.zeros((n,), dtype=np.int64)
    for (i, j, k) in trip:
        m = fill[i]
        jj[m, i] = j
        kk[m, i] = k
        fill[i] += 1
    assert (fill == _SLOTS).all()
    return jj, kk, trip.shape[0]


_JJ, _KK, _NUM_TRIPLETS = _triplet_columns(_N)
_IDX = np.zeros((8, _N), dtype=np.int32)
_IDX[0:3] = _JJ
_IDX[4:7] = _KK


def _loss_kernel(x_ref, idx_ref, out_ref, acc_ref):
    d = pl.program_id(0)
    xc = x_ref[...].astype(jnp.bfloat16)
    part = jax.lax.dot_general(
        xc, xc,
        dimension_numbers=(((1,), (1,)), ((), ())),
        preferred_element_type=jnp.float32,
    )

    @pl.when(d == 0)
    def _init():
        acc_ref[...] = part

    @pl.when(d > 0)
    def _accum():
        acc_ref[...] += part

    @pl.when(d == _NCHUNKS - 1)
    def _epilogue():
        gram = acc_ref[...]
        rows = jax.lax.broadcasted_iota(jnp.int32, (_N, _N), 0)
        cols = jax.lax.broadcasted_iota(jnp.int32, (_N, _N), 1)
        diag = jnp.where(rows == cols, gram, 0.0)
        xn_col = jnp.sum(diag, axis=1, keepdims=True)
        xn_row = jnp.sum(diag, axis=0, keepdims=True)
        dist = jnp.maximum(xn_col + xn_row - 2.0 * gram, 0.0)

        total = jnp.zeros((), dtype=jnp.float32)
        for m in range(_SLOTS):
            jj = idx_ref[m, :].reshape(_N, 1)
            kk = idx_ref[4 + m, :].reshape(_N, 1)
            sel = jnp.where(cols == jj, dist, 0.0) - jnp.where(cols == kk, dist, 0.0)
            delta = jnp.sum(sel, axis=1)
            per = jnp.maximum(delta, 0.0) + jnp.log1p(jnp.exp(-jnp.abs(delta)))
            total = total + jnp.sum(per)
        out_ref[...] = jnp.reshape(total / float(_NUM_TRIPLETS), (1, 1))


@jax.jit
def kernel(x):
    idx = jnp.asarray(_IDX)
    out = pl.pallas_call(
        _loss_kernel,
        grid=(_NCHUNKS,),
        out_shape=jax.ShapeDtypeStruct((1, 1), jnp.float32),
        in_specs=[
            pl.BlockSpec((_N, _CHUNK), lambda d: (0, d)),
            pl.BlockSpec((8, _N), lambda d: (0, 0)),
        ],
        out_specs=pl.BlockSpec((1, 1), lambda d: (0, 0)),
        scratch_shapes=[pltpu.VMEM((_N, _N), jnp.float32)],
    )(x, idx)
    return out.reshape((1,))


# speedup vs baseline: 1.4243x; 1.4243x over previous
import numpy as np
import jax
import jax.numpy as jnp
from jax.experimental import pallas as pl
from jax.experimental.pallas import tpu as pltpu

_N = 512
_D = 4096
_CHUNK = 2048
_NCHUNKS = _D // _CHUNK
_SLOTS = 3


def _triplet_columns(n):
    labels = list(range(int(n / 2))) + list(range(int(n / 2)))
    rng = np.random.RandomState(0)
    triplets = []
    for i in range(len(labels)):
        triplets_i = []
        for j in range(len(labels)):
            if labels[i] == labels[j] and i != j:
                for k in range(len(labels)):
                    if labels[i] != labels[k]:
                        triplets_i.append([i, j, k])
        rng.shuffle(triplets_i)
        triplets += triplets_i[:3]
    trip = np.asarray(triplets, dtype=np.int32)
    jj = np.zeros((_SLOTS, n), dtype=np.int32)
    kk = np.zeros((_SLOTS, n), dtype=np.int32)
    fill = np.zeros((n,), dtype=np.int64)
    for (i, j, k) in trip:
        m = fill[i]
        jj[m, i] = j
        kk[m, i] = k
        fill[i] += 1
    assert (fill == _SLOTS).all()
    return jj, kk, trip.shape[0]


_JJ, _KK, _NUM_TRIPLETS = _triplet_columns(_N)
_IDX = np.zeros((8, _N), dtype=np.int32)
_IDX[0:3] = _JJ
_IDX[4:7] = _KK


def _loss_kernel(x_ref, idx_ref, out_ref, acc_ref):
    d = pl.program_id(0)
    xc = x_ref[...].astype(jnp.bfloat16)
    part = jax.lax.dot_general(
        xc, xc,
        dimension_numbers=(((1,), (1,)), ((), ())),
        preferred_element_type=jnp.float32,
    )

    @pl.when(d == 0)
    def _init():
        acc_ref[...] = part

    @pl.when(d > 0)
    def _accum():
        acc_ref[...] += part

    @pl.when(d == _NCHUNKS - 1)
    def _epilogue():
        gram = acc_ref[...]
        rows = jax.lax.broadcasted_iota(jnp.int32, (_N, _N), 0)
        cols = jax.lax.broadcasted_iota(jnp.int32, (_N, _N), 1)
        diag = jnp.where(rows == cols, gram, 0.0)
        xn_col = jnp.sum(diag, axis=1, keepdims=True)
        xn_row = jnp.sum(diag, axis=0, keepdims=True)
        dist = jnp.maximum(xn_col + xn_row - 2.0 * gram, 0.0)

        total = jnp.zeros((), dtype=jnp.float32)
        for m in range(_SLOTS):
            jj = idx_ref[m, :].reshape(_N, 1)
            kk = idx_ref[4 + m, :].reshape(_N, 1)
            sel = jnp.where(cols == jj, dist, 0.0) - jnp.where(cols == kk, dist, 0.0)
            delta = jnp.sum(sel, axis=1)
            per = jnp.maximum(delta, 0.0) + jnp.log1p(jnp.exp(-jnp.abs(delta)))
            total = total + jnp.sum(per)
        out_ref[...] = jnp.reshape(total / float(_NUM_TRIPLETS), (1, 1))


@jax.jit
def kernel(x):
    idx = jnp.asarray(_IDX)
    out = pl.pallas_call(
        _loss_kernel,
        grid=(_NCHUNKS,),
        out_shape=jax.ShapeDtypeStruct((1, 1), jnp.float32),
        in_specs=[
            pl.BlockSpec((_N, _CHUNK), lambda d: (0, d)),
            pl.BlockSpec((8, _N), lambda d: (0, 0)),
        ],
        out_specs=pl.BlockSpec((1, 1), lambda d: (0, 0)),
        scratch_shapes=[pltpu.VMEM((_N, _N), jnp.float32)],
    )(x, idx)
    return out.reshape((1,))
